# baseline (speedup 1.0000x reference)
import functools

import jax
import jax.numpy as jnp
from jax import lax
from jax.experimental import pallas as pl
from jax.experimental.pallas import tpu as pltpu
from jax.experimental.pallas import tpu_sc as plsc

N = 10000
E = 320000
D_IN, D_HID, D_OUT = 128, 64, 6
NC, NS = 2, 16
NW = NC * NS
N_PAD = 10240
ROWS_T = N_PAD // NS
E_W = E // NW
_CHUNK_CFG = {64: (200, 5, 3), 16: (500, 5, 3)}
BN = 1024
NB = N_PAD // BN


def _ring_loop(rows, acc, src_v, dst_v, bufs, gsem, ssem, B, U, G, NCH):
    def gdesc(k, j):
        return pltpu.make_async_copy(rows.at[src_v.at[k]], bufs[j], gsem[j])

    def sdesc(k, j):
        return pltpu.make_async_copy(bufs[j], acc.at[dst_v.at[k]], ssem[j])

    for j in range(G):
        gdesc(j, j).start()

    def step(i, carry):
        for j in range(U):
            k = U * i + j
            gdesc(k, j).wait()
            sdesc(k, j).start(add=True)
            jj = (j + G) % U

            @pl.when(k >= U - G)
            def _():
                sdesc(k - (U - G), jj).wait()

            @pl.when(k + G < NCH)
            def _():
                gdesc(k + G, jj).start()

        return carry

    lax.fori_loop(0, NCH // U, step, 0)
    for t in range(U - G):
        k = NCH - (U - G) + t
        sdesc(k, k % U).wait()


@functools.lru_cache(maxsize=None)
def _make_spass(d):
    B, U, G = _CHUNK_CFG[d]
    NCH = E_W // B
    assert NCH % U == 0 and E_W % B == 0
    mesh = plsc.VectorSubcoreMesh(
        core_axis_name="c", subcore_axis_name="s", num_cores=NC, num_subcores=NS
    )

    @functools.partial(
        pl.kernel,
        out_type=jax.ShapeDtypeStruct((NC * N_PAD, d), jnp.float32),
        mesh=mesh,
        compiler_params=pltpu.CompilerParams(use_tc_tiling_on_sc=False),
        scratch_types=[
            pltpu.VMEM((NCH, B), jnp.int32),
            pltpu.VMEM((NCH, B), jnp.int32),
            *([pltpu.VMEM((B, d), jnp.float32)] * U),
            *([pltpu.SemaphoreType.DMA] * U),
            *([pltpu.SemaphoreType.DMA] * U),
            pltpu.VMEM_SHARED((N_PAD, d), jnp.float32),
        ],
    )
    def spass(src3, dst3, rows, zeros, out, src_v, dst_v, *rest):
        bufs = rest[:U]
        gsem = rest[U:2 * U]
        ssem = rest[2 * U:3 * U]
        acc = rest[3 * U]
        c = lax.axis_index("c")
        s = lax.axis_index("s")
        wid = c * NS + s
        row0 = s * ROWS_T

        pltpu.sync_copy(zeros.at[pl.ds(row0, ROWS_T)], acc.at[pl.ds(row0, ROWS_T)])
        pltpu.sync_copy(src3.at[wid], src_v)
        pltpu.sync_copy(dst3.at[wid], dst_v)
        plsc.subcore_barrier()
        _ring_loop(rows, acc, src_v, dst_v, bufs, gsem, ssem, B, U, G, NCH)
        plsc.subcore_barrier()
        pltpu.sync_copy(
            acc.at[pl.ds(row0, ROWS_T)],
            out.at[pl.ds(c * N_PAD + row0, ROWS_T)],
        )

    return spass


_R3C = 128


@functools.lru_cache(maxsize=None)
def _make_spass2(d):
    B, U, G = _CHUNK_CFG[d]
    NCH = E_W // B
    assert NCH % U == 0 and E_W % B == 0 and ROWS_T % _R3C == 0 and B >= _R3C
    mesh = plsc.VectorSubcoreMesh(
        core_axis_name="c", subcore_axis_name="s", num_cores=NC, num_subcores=NS
    )

    @functools.partial(
        pl.kernel,
        out_type=(jax.ShapeDtypeStruct((NC * N_PAD, d), jnp.float32),
                  jax.ShapeDtypeStruct((N_PAD, d), jnp.float32)),
        mesh=mesh,
        compiler_params=pltpu.CompilerParams(
            use_tc_tiling_on_sc=False, needs_layout_passes=False
        ),
        scratch_types=[
            pltpu.VMEM((NCH, B), jnp.int32),
            pltpu.VMEM((NCH, B), jnp.int32),
            *([pltpu.VMEM((B, d), jnp.float32)] * U),
            *([pltpu.SemaphoreType.DMA] * U),
            *([pltpu.SemaphoreType.DMA] * U),
            pltpu.VMEM((ROWS_T,), jnp.float32),
            pltpu.VMEM_SHARED((N_PAD, d), jnp.float32),
        ],
    )
    def spass2(src3, dst3, pp, u, dinv, zeros, out, u2, src_v, dst_v, *rest):
        bufs = rest[:U]
        gsem = rest[U:2 * U]
        ssem = rest[2 * U:3 * U]
        dinv_v = rest[3 * U]
        acc = rest[3 * U + 1]
        c = lax.axis_index("c")
        s = lax.axis_index("s")
        wid = c * NS + s
        row0 = s * ROWS_T

        pltpu.sync_copy(zeros.at[pl.ds(row0, ROWS_T)], acc.at[pl.ds(row0, ROWS_T)])
        pltpu.sync_copy(dinv.at[pl.ds(row0, ROWS_T)], dinv_v)
        pltpu.sync_copy(src3.at[wid], src_v)
        pltpu.sync_copy(dst3.at[wid], dst_v)

        for t in range(ROWS_T // _R3C):
            r0 = row0 + t * _R3C
            cps = [
                pltpu.make_async_copy(pp.at[pl.ds(r0, _R3C)],
                                      bufs[0].at[pl.ds(0, _R3C)], gsem[0]),
                pltpu.make_async_copy(pp.at[pl.ds(N_PAD + r0, _R3C)],
                                      bufs[1].at[pl.ds(0, _R3C)], gsem[1]),
                pltpu.make_async_copy(u.at[pl.ds(r0, _R3C)],
                                      bufs[2].at[pl.ds(0, _R3C)], gsem[2]),
            ]
            for cp in cps:
                cp.start()
            for cp in cps:
                cp.wait()

            def rowbody(r, carry, _t=t):
                sp = plsc.load_gather(
                    dinv_v, [jnp.full((16,), _t * _R3C + r, jnp.int32)]
                )
                sq = sp * sp
                for j in range(d // 16):
                    sl = pl.ds(j * 16, 16)
                    bufs[3][r, sl] = sq * (
                        bufs[0][r, sl] + bufs[1][r, sl] + bufs[2][r, sl]
                    )
                return carry

            lax.fori_loop(0, _R3C, rowbody, 0)
            pltpu.sync_copy(bufs[3].at[pl.ds(0, _R3C)], u2.at[pl.ds(r0, _R3C)])

        plsc.subcore_barrier()
        _ring_loop(u2, acc, src_v, dst_v, bufs, gsem, ssem, B, U, G, NCH)
        plsc.subcore_barrier()
        pltpu.sync_copy(
            acc.at[pl.ds(row0, ROWS_T)],
            out.at[pl.ds(c * N_PAD + row0, ROWS_T)],
        )

    return spass2


def _row_spec(d, shift=0):
    return pl.BlockSpec((BN, d), lambda i, _s=shift: (i + _s, 0))


def _full_spec(shape):
    nd = len(shape)
    return pl.BlockSpec(shape, lambda i, _nd=nd: (0,) * nd)


def _tc_call(body, in_specs, out_shapes, out_specs):
    return pl.pallas_call(
        body,
        grid=(NB,),
        in_specs=in_specs,
        out_shape=out_shapes,
        out_specs=out_specs,
    )


def _tc_a(degp, x, w1):
    def body(p0_ref, p1_ref, x_ref, w1_ref, dinv_ref, u_ref):
        deg = 1.0 + p0_ref[:, 0:1] + p1_ref[:, 0:1]
        dinv = lax.rsqrt(deg)
        y1 = jnp.dot(x_ref[...], w1_ref[...], preferred_element_type=jnp.float32)
        dinv_ref[...] = dinv
        u_ref[...] = dinv * y1

    return _tc_call(
        body,
        [_row_spec(16), _row_spec(16, NB), _row_spec(D_IN), _full_spec((D_IN, D_HID))],
        (jax.ShapeDtypeStruct((N_PAD, 1), jnp.float32),
         jax.ShapeDtypeStruct((N_PAD, D_HID), jnp.float32)),
        (_row_spec(1), _row_spec(D_HID)),
    )(degp, degp, x, w1)


def _tc_mid(s, u, dinv, d):
    def body(s0_ref, s1_ref, u_ref, dinv_ref, o_ref):
        dinv = dinv_ref[...]
        o_ref[...] = dinv * dinv * (s0_ref[...] + s1_ref[...] + u_ref[...])

    return _tc_call(
        body,
        [_row_spec(d), _row_spec(d, NB), _row_spec(d), _row_spec(1)],
        jax.ShapeDtypeStruct((N_PAD, d), jnp.float32),
        _row_spec(d),
    )(s, s, u, dinv)


def _tc_c(s, u2, dinv, b1, w2p):
    def body(s0_ref, s1_ref, u2_ref, dinv_ref, b1_ref, w2_ref, o_ref):
        dinv = dinv_ref[...]
        t = dinv * (s0_ref[...] + s1_ref[...] + u2_ref[...])
        h = jnp.maximum(t + b1_ref[...], 0.0)
        y2 = jnp.dot(h, w2_ref[...], preferred_element_type=jnp.float32)
        o_ref[...] = dinv * y2

    return _tc_call(
        body,
        [_row_spec(D_HID), _row_spec(D_HID, NB), _row_spec(D_HID), _row_spec(1),
         _full_spec((1, D_HID)), _full_spec((D_HID, 16))],
        jax.ShapeDtypeStruct((N_PAD, 16), jnp.float32),
        _row_spec(16),
    )(s, s, u2, dinv, b1, w2p)


def _tc_e(s, u4, dinv, b2p):
    def body(s0_ref, s1_ref, u4_ref, dinv_ref, b2_ref, o_ref):
        dinv = dinv_ref[...]
        o = dinv * (s0_ref[...] + s1_ref[...] + u4_ref[...]) + b2_ref[...]
        cols = lax.broadcasted_iota(jnp.int32, (BN, 16), 1)
        valid = cols < D_OUT
        m = jnp.max(jnp.where(valid, o, -jnp.inf), axis=1, keepdims=True)
        e = jnp.where(valid, jnp.exp(o - m), 0.0)
        lse = m + jnp.log(jnp.sum(e, axis=1, keepdims=True))
        o_ref[...] = o - lse

    return _tc_call(
        body,
        [_row_spec(16), _row_spec(16, NB), _row_spec(16), _row_spec(1),
         _full_spec((1, 16))],
        jax.ShapeDtypeStruct((N_PAD, 16), jnp.float32),
        _row_spec(16),
    )(s, s, u4, dinv, b2p)


def kernel(x, edge_index, W1, b1, W2, b2):
    def idx3(d):
        B, _, _ = _CHUNK_CFG[d]
        shape = (NW, E_W // B, B)
        return edge_index[0].reshape(shape), edge_index[1].reshape(shape)

    src64, dst64 = idx3(64)
    src16, dst16 = idx3(16)
    xp = jnp.zeros((N_PAD, D_IN), jnp.float32).at[:N].set(x)
    zeros64 = jnp.zeros((N_PAD, D_HID), jnp.float32)
    zeros16 = jnp.zeros((N_PAD, 16), jnp.float32)
    ones16 = jnp.ones((N_PAD, 16), jnp.float32)
    w2p = jnp.zeros((D_HID, 16), jnp.float32).at[:, :D_OUT].set(W2)
    b2p = jnp.zeros((1, 16), jnp.float32).at[0, :D_OUT].set(b2)

    spass64 = _make_spass(D_HID)
    spass16 = _make_spass(16)
    spass2_64 = _make_spass2(D_HID)
    spass2_16 = _make_spass2(16)

    degp = spass16(src16, dst16, ones16, zeros16)
    dinv, u = _tc_a(degp, xp, W1)
    dinv1d = dinv.reshape(N_PAD)
    s1 = spass64(src64, dst64, u, zeros64)
    s2, u2 = spass2_64(src64, dst64, s1, u, dinv1d, zeros64)
    u3 = _tc_c(s2, u2, dinv, b1.reshape(1, D_HID), w2p)
    s3 = spass16(src16, dst16, u3, zeros16)
    s4, u4 = spass2_16(src16, dst16, s3, u3, dinv1d, zeros16)
    out = _tc_e(s4, u4, dinv, b2p)
    return out[:N, :D_OUT]


# speedup vs baseline: 1.0863x; 1.0863x over previous
import functools

import jax
import jax.numpy as jnp
from jax import lax
from jax.experimental import pallas as pl
from jax.experimental.pallas import tpu as pltpu
from jax.experimental.pallas import tpu_sc as plsc

N = 10000
E = 320000
D_IN, D_HID, D_OUT = 128, 64, 6
NC, NS = 2, 16
NW = NC * NS
N_PAD = 10240
ROWS_T = N_PAD // NS
E_W = E // NW
_CHUNK_CFG = {64: (200, 5, 3), 16: (500, 5, 3)}
BN = 1024
NB = N_PAD // BN


def _ring_loop(rows, acc, src_v, dst_v, bufs, gsem, ssem, B, U, G, NCH):
    def gdesc(k, j):
        return pltpu.make_async_copy(rows.at[src_v.at[k]], bufs[j], gsem[j])

    def sdesc(k, j):
        return pltpu.make_async_copy(bufs[j], acc.at[dst_v.at[k]], ssem[j])

    for j in range(G):
        gdesc(j, j).start()

    def step(i, carry):
        for j in range(U):
            k = U * i + j
            gdesc(k, j).wait()
            sdesc(k, j).start(add=True)
            jj = (j + G) % U

            @pl.when(k >= U - G)
            def _():
                sdesc(k - (U - G), jj).wait()

            @pl.when(k + G < NCH)
            def _():
                gdesc(k + G, jj).start()

        return carry

    lax.fori_loop(0, NCH // U, step, 0)
    for t in range(U - G):
        k = NCH - (U - G) + t
        sdesc(k, k % U).wait()


@functools.lru_cache(maxsize=None)
def _make_spass(d):
    B, U, G = _CHUNK_CFG[d]
    NCH = E_W // B
    assert NCH % U == 0 and E_W % B == 0
    mesh = plsc.VectorSubcoreMesh(
        core_axis_name="c", subcore_axis_name="s", num_cores=NC, num_subcores=NS
    )

    @functools.partial(
        pl.kernel,
        out_type=jax.ShapeDtypeStruct((NC * N_PAD, d), jnp.float32),
        mesh=mesh,
        compiler_params=pltpu.CompilerParams(use_tc_tiling_on_sc=False),
        scratch_types=[
            pltpu.VMEM((NCH, B), jnp.int32),
            pltpu.VMEM((NCH, B), jnp.int32),
            *([pltpu.VMEM((B, d), jnp.float32)] * U),
            *([pltpu.SemaphoreType.DMA] * U),
            *([pltpu.SemaphoreType.DMA] * U),
            pltpu.VMEM_SHARED((N_PAD, d), jnp.float32),
        ],
    )
    def spass(src3, dst3, rows, zeros, out, src_v, dst_v, *rest):
        bufs = rest[:U]
        gsem = rest[U:2 * U]
        ssem = rest[2 * U:3 * U]
        acc = rest[3 * U]
        c = lax.axis_index("c")
        s = lax.axis_index("s")
        wid = c * NS + s
        row0 = s * ROWS_T

        pltpu.sync_copy(zeros.at[pl.ds(row0, ROWS_T)], acc.at[pl.ds(row0, ROWS_T)])
        pltpu.sync_copy(src3.at[wid], src_v)
        pltpu.sync_copy(dst3.at[wid], dst_v)
        plsc.subcore_barrier()
        _ring_loop(rows, acc, src_v, dst_v, bufs, gsem, ssem, B, U, G, NCH)
        plsc.subcore_barrier()
        pltpu.sync_copy(
            acc.at[pl.ds(row0, ROWS_T)],
            out.at[pl.ds(c * N_PAD + row0, ROWS_T)],
        )

    return spass


_R3C = 128


@functools.lru_cache(maxsize=None)
def _make_spass2(d):
    B, U, G = _CHUNK_CFG[d]
    NCH = E_W // B
    assert NCH % U == 0 and E_W % B == 0 and ROWS_T % _R3C == 0 and B >= _R3C
    mesh = plsc.VectorSubcoreMesh(
        core_axis_name="c", subcore_axis_name="s", num_cores=NC, num_subcores=NS
    )

    @functools.partial(
        pl.kernel,
        out_type=(jax.ShapeDtypeStruct((NC * N_PAD, d), jnp.float32),
                  jax.ShapeDtypeStruct((N_PAD, d), jnp.float32)),
        mesh=mesh,
        compiler_params=pltpu.CompilerParams(
            use_tc_tiling_on_sc=False, needs_layout_passes=False
        ),
        scratch_types=[
            pltpu.VMEM((NCH, B), jnp.int32),
            pltpu.VMEM((NCH, B), jnp.int32),
            *([pltpu.VMEM((B, d), jnp.float32)] * U),
            *([pltpu.SemaphoreType.DMA] * U),
            *([pltpu.SemaphoreType.DMA] * U),
            pltpu.VMEM((ROWS_T,), jnp.float32),
            pltpu.VMEM_SHARED((N_PAD, d), jnp.float32),
        ],
    )
    def spass2(src3, dst3, pp, u, dinv, zeros, out, u2, src_v, dst_v, *rest):
        bufs = rest[:U]
        gsem = rest[U:2 * U]
        ssem = rest[2 * U:3 * U]
        dinv_v = rest[3 * U]
        acc = rest[3 * U + 1]
        c = lax.axis_index("c")
        s = lax.axis_index("s")
        wid = c * NS + s
        row0 = s * ROWS_T

        pltpu.sync_copy(zeros.at[pl.ds(row0, ROWS_T)], acc.at[pl.ds(row0, ROWS_T)])
        pltpu.sync_copy(dinv.at[pl.ds(row0, ROWS_T)], dinv_v)
        pltpu.sync_copy(src3.at[wid], src_v)
        pltpu.sync_copy(dst3.at[wid], dst_v)

        for t in range(ROWS_T // _R3C):
            r0 = row0 + t * _R3C
            cps = [
                pltpu.make_async_copy(pp.at[pl.ds(r0, _R3C)],
                                      bufs[0].at[pl.ds(0, _R3C)], gsem[0]),
                pltpu.make_async_copy(pp.at[pl.ds(N_PAD + r0, _R3C)],
                                      bufs[1].at[pl.ds(0, _R3C)], gsem[1]),
                pltpu.make_async_copy(u.at[pl.ds(r0, _R3C)],
                                      bufs[2].at[pl.ds(0, _R3C)], gsem[2]),
            ]
            for cp in cps:
                cp.start()
            for cp in cps:
                cp.wait()

            def rowbody(r, carry, _t=t):
                sp = plsc.load_gather(
                    dinv_v, [jnp.full((16,), _t * _R3C + r, jnp.int32)]
                )
                sq = sp * sp
                for j in range(d // 16):
                    sl = pl.ds(j * 16, 16)
                    bufs[3][r, sl] = sq * (
                        bufs[0][r, sl] + bufs[1][r, sl] + bufs[2][r, sl]
                    )
                return carry

            lax.fori_loop(0, _R3C, rowbody, 0)
            pltpu.sync_copy(bufs[3].at[pl.ds(0, _R3C)], u2.at[pl.ds(r0, _R3C)])

        plsc.subcore_barrier()
        _ring_loop(u2, acc, src_v, dst_v, bufs, gsem, ssem, B, U, G, NCH)
        plsc.subcore_barrier()
        pltpu.sync_copy(
            acc.at[pl.ds(row0, ROWS_T)],
            out.at[pl.ds(c * N_PAD + row0, ROWS_T)],
        )

    return spass2


ROWS_W = N_PAD // NW


@functools.lru_cache(maxsize=None)
def _make_fused(d):
    B, U, G = _CHUNK_CFG[d]
    NCH = E_W // B
    R3C = 160
    assert NCH % U == 0 and ROWS_W % R3C == 0 and B >= R3C
    mesh = plsc.VectorSubcoreMesh(
        core_axis_name="c", subcore_axis_name="s", num_cores=NC, num_subcores=NS
    )

    @functools.partial(
        pl.kernel,
        out_type=(jax.ShapeDtypeStruct((NC * N_PAD, d), jnp.float32),
                  jax.ShapeDtypeStruct((N_PAD, d), jnp.float32)),
        mesh=mesh,
        compiler_params=pltpu.CompilerParams(
            use_tc_tiling_on_sc=False, needs_layout_passes=False
        ),
        scratch_types=[
            pltpu.VMEM((NCH, B), jnp.int32),
            pltpu.VMEM((NCH, B), jnp.int32),
            *([pltpu.VMEM((B, d), jnp.float32)] * U),
            *([pltpu.SemaphoreType.DMA] * U),
            *([pltpu.SemaphoreType.DMA] * U),
            pltpu.VMEM((ROWS_W,), jnp.float32),
            pltpu.SemaphoreType.REGULAR,
            pltpu.VMEM_SHARED((N_PAD, d), jnp.float32),
        ],
    )
    def fused(src3, dst3, u, dinv, zeros, out, u2, src_v, dst_v, *rest):
        bufs = rest[:U]
        gsem = rest[U:2 * U]
        ssem = rest[2 * U:3 * U]
        dinv_v = rest[3 * U]
        xsem = rest[3 * U + 1]
        acc = rest[3 * U + 2]
        c = lax.axis_index("c")
        s = lax.axis_index("s")
        wid = c * NS + s
        row0 = s * ROWS_T
        grow0 = wid * ROWS_W

        def cross_core_barrier():
            plsc.subcore_barrier()
            pl.semaphore_signal(xsem, 1, core_index=1 - c)
            pl.semaphore_wait(xsem, 1)

        pltpu.sync_copy(zeros.at[pl.ds(row0, ROWS_T)], acc.at[pl.ds(row0, ROWS_T)])
        pltpu.sync_copy(dinv.at[pl.ds(grow0, ROWS_W)], dinv_v)
        pltpu.sync_copy(src3.at[wid], src_v)
        pltpu.sync_copy(dst3.at[wid], dst_v)
        plsc.subcore_barrier()
        _ring_loop(u, acc, src_v, dst_v, bufs, gsem, ssem, B, U, G, NCH)
        plsc.subcore_barrier()
        pltpu.sync_copy(
            acc.at[pl.ds(row0, ROWS_T)],
            out.at[pl.ds(c * N_PAD + row0, ROWS_T)],
        )
        cross_core_barrier()

        for t in range(ROWS_W // R3C):
            r0 = grow0 + t * R3C
            cps = [
                pltpu.make_async_copy(out.at[pl.ds(r0, R3C)],
                                      bufs[0].at[pl.ds(0, R3C)], gsem[0]),
                pltpu.make_async_copy(out.at[pl.ds(N_PAD + r0, R3C)],
                                      bufs[1].at[pl.ds(0, R3C)], gsem[1]),
                pltpu.make_async_copy(u.at[pl.ds(r0, R3C)],
                                      bufs[2].at[pl.ds(0, R3C)], gsem[2]),
            ]
            for cp in cps:
                cp.start()
            for cp in cps:
                cp.wait()

            def rowbody(r, carry, _t=t):
                sp = plsc.load_gather(
                    dinv_v, [jnp.full((16,), _t * R3C + r, jnp.int32)]
                )
                sq = sp * sp
                for j in range(d // 16):
                    sl = pl.ds(j * 16, 16)
                    bufs[3][r, sl] = sq * (
                        bufs[0][r, sl] + bufs[1][r, sl] + bufs[2][r, sl]
                    )
                return carry

            lax.fori_loop(0, R3C, rowbody, 0)
            pltpu.sync_copy(bufs[3].at[pl.ds(0, R3C)], u2.at[pl.ds(r0, R3C)])

        cross_core_barrier()

        pltpu.sync_copy(zeros.at[pl.ds(row0, ROWS_T)], acc.at[pl.ds(row0, ROWS_T)])
        plsc.subcore_barrier()
        _ring_loop(u2, acc, src_v, dst_v, bufs, gsem, ssem, B, U, G, NCH)
        plsc.subcore_barrier()
        pltpu.sync_copy(
            acc.at[pl.ds(row0, ROWS_T)],
            out.at[pl.ds(c * N_PAD + row0, ROWS_T)],
        )

    return fused


def _row_spec(d, shift=0):
    return pl.BlockSpec((BN, d), lambda i, _s=shift: (i + _s, 0))


def _full_spec(shape):
    nd = len(shape)
    return pl.BlockSpec(shape, lambda i, _nd=nd: (0,) * nd)


def _tc_call(body, in_specs, out_shapes, out_specs):
    return pl.pallas_call(
        body,
        grid=(NB,),
        in_specs=in_specs,
        out_shape=out_shapes,
        out_specs=out_specs,
    )


def _tc_a(degp, x, w1):
    def body(p0_ref, p1_ref, x_ref, w1_ref, dinv_ref, u_ref):
        deg = 1.0 + p0_ref[:, 0:1] + p1_ref[:, 0:1]
        dinv = lax.rsqrt(deg)
        y1 = jnp.dot(x_ref[...], w1_ref[...], preferred_element_type=jnp.float32)
        dinv_ref[...] = dinv
        u_ref[...] = dinv * y1

    return _tc_call(
        body,
        [_row_spec(16), _row_spec(16, NB), _row_spec(D_IN), _full_spec((D_IN, D_HID))],
        (jax.ShapeDtypeStruct((N_PAD, 1), jnp.float32),
         jax.ShapeDtypeStruct((N_PAD, D_HID), jnp.float32)),
        (_row_spec(1), _row_spec(D_HID)),
    )(degp, degp, x, w1)


def _tc_mid(s, u, dinv, d):
    def body(s0_ref, s1_ref, u_ref, dinv_ref, o_ref):
        dinv = dinv_ref[...]
        o_ref[...] = dinv * dinv * (s0_ref[...] + s1_ref[...] + u_ref[...])

    return _tc_call(
        body,
        [_row_spec(d), _row_spec(d, NB), _row_spec(d), _row_spec(1)],
        jax.ShapeDtypeStruct((N_PAD, d), jnp.float32),
        _row_spec(d),
    )(s, s, u, dinv)


def _tc_c(s, u2, dinv, b1, w2p):
    def body(s0_ref, s1_ref, u2_ref, dinv_ref, b1_ref, w2_ref, o_ref):
        dinv = dinv_ref[...]
        t = dinv * (s0_ref[...] + s1_ref[...] + u2_ref[...])
        h = jnp.maximum(t + b1_ref[...], 0.0)
        y2 = jnp.dot(h, w2_ref[...], preferred_element_type=jnp.float32)
        o_ref[...] = dinv * y2

    return _tc_call(
        body,
        [_row_spec(D_HID), _row_spec(D_HID, NB), _row_spec(D_HID), _row_spec(1),
         _full_spec((1, D_HID)), _full_spec((D_HID, 16))],
        jax.ShapeDtypeStruct((N_PAD, 16), jnp.float32),
        _row_spec(16),
    )(s, s, u2, dinv, b1, w2p)


def _tc_e(s, u4, dinv, b2p):
    def body(s0_ref, s1_ref, u4_ref, dinv_ref, b2_ref, o_ref):
        dinv = dinv_ref[...]
        o = dinv * (s0_ref[...] + s1_ref[...] + u4_ref[...]) + b2_ref[...]
        cols = lax.broadcasted_iota(jnp.int32, (BN, 16), 1)
        valid = cols < D_OUT
        m = jnp.max(jnp.where(valid, o, -jnp.inf), axis=1, keepdims=True)
        e = jnp.where(valid, jnp.exp(o - m), 0.0)
        lse = m + jnp.log(jnp.sum(e, axis=1, keepdims=True))
        o_ref[...] = o - lse

    return _tc_call(
        body,
        [_row_spec(16), _row_spec(16, NB), _row_spec(16), _row_spec(1),
         _full_spec((1, 16))],
        jax.ShapeDtypeStruct((N_PAD, 16), jnp.float32),
        _row_spec(16),
    )(s, s, u4, dinv, b2p)


def kernel(x, edge_index, W1, b1, W2, b2):
    def idx3(d):
        B, _, _ = _CHUNK_CFG[d]
        shape = (NW, E_W // B, B)
        return edge_index[0].reshape(shape), edge_index[1].reshape(shape)

    src64, dst64 = idx3(64)
    src16, dst16 = idx3(16)
    xp = jnp.zeros((N_PAD, D_IN), jnp.float32).at[:N].set(x)
    zeros64 = jnp.zeros((N_PAD, D_HID), jnp.float32)
    zeros16 = jnp.zeros((N_PAD, 16), jnp.float32)
    ones16 = jnp.ones((N_PAD, 16), jnp.float32)
    w2p = jnp.zeros((D_HID, 16), jnp.float32).at[:, :D_OUT].set(W2)
    b2p = jnp.zeros((1, 16), jnp.float32).at[0, :D_OUT].set(b2)

    spass16 = _make_spass(16)
    fused64 = _make_fused(D_HID)
    fused16 = _make_fused(16)

    degp = spass16(src16, dst16, ones16, zeros16)
    dinv, u = _tc_a(degp, xp, W1)
    dinv1d = dinv.reshape(N_PAD)
    s2, u2 = fused64(src64, dst64, u, dinv1d, zeros64)
    u3 = _tc_c(s2, u2, dinv, b1.reshape(1, D_HID), w2p)
    s4, u4 = fused16(src16, dst16, u3, dinv1d, zeros16)
    out = _tc_e(s4, u4, dinv, b2p)
    return out[:N, :D_OUT]


# speedup vs baseline: 1.0877x; 1.0012x over previous
import functools

import jax
import jax.numpy as jnp
from jax import lax
from jax.experimental import pallas as pl
from jax.experimental.pallas import tpu as pltpu
from jax.experimental.pallas import tpu_sc as plsc

N = 10000
E = 320000
D_IN, D_HID, D_OUT = 128, 64, 6
NC, NS = 2, 16
NW = NC * NS
N_PAD = 10240
ROWS_T = N_PAD // NS
E_W = E // NW
_CHUNK_CFG = {64: (200, 5, 3), 16: (500, 5, 3)}
BN = 1024
NB = N_PAD // BN


def _ring_loop(rows, acc, src_v, dst_v, bufs, gsem, ssem, B, U, G, NCH):
    def gdesc(k, j):
        return pltpu.make_async_copy(rows.at[src_v.at[k]], bufs[j], gsem[j])

    def sdesc(k, j):
        return pltpu.make_async_copy(bufs[j], acc.at[dst_v.at[k]], ssem[j])

    for j in range(G):
        gdesc(j, j).start()

    def step(i, carry):
        for j in range(U):
            k = U * i + j
            gdesc(k, j).wait()
            sdesc(k, j).start(add=True)
            jj = (j + G) % U

            @pl.when(k >= U - G)
            def _():
                sdesc(k - (U - G), jj).wait()

            @pl.when(k + G < NCH)
            def _():
                gdesc(k + G, jj).start()

        return carry

    lax.fori_loop(0, NCH // U, step, 0)
    for t in range(U - G):
        k = NCH - (U - G) + t
        sdesc(k, k % U).wait()


@functools.lru_cache(maxsize=None)
def _make_spass(d):
    B, U, G = _CHUNK_CFG[d]
    NCH = E_W // B
    assert NCH % U == 0 and E_W % B == 0
    mesh = plsc.VectorSubcoreMesh(
        core_axis_name="c", subcore_axis_name="s", num_cores=NC, num_subcores=NS
    )

    @functools.partial(
        pl.kernel,
        out_type=jax.ShapeDtypeStruct((NC * N_PAD, d), jnp.float32),
        mesh=mesh,
        compiler_params=pltpu.CompilerParams(use_tc_tiling_on_sc=False),
        scratch_types=[
            pltpu.VMEM((NCH, B), jnp.int32),
            pltpu.VMEM((NCH, B), jnp.int32),
            *([pltpu.VMEM((B, d), jnp.float32)] * U),
            *([pltpu.SemaphoreType.DMA] * U),
            *([pltpu.SemaphoreType.DMA] * U),
            pltpu.VMEM_SHARED((N_PAD, d), jnp.float32),
        ],
    )
    def spass(src3, dst3, rows, zeros, out, src_v, dst_v, *rest):
        bufs = rest[:U]
        gsem = rest[U:2 * U]
        ssem = rest[2 * U:3 * U]
        acc = rest[3 * U]
        c = lax.axis_index("c")
        s = lax.axis_index("s")
        wid = c * NS + s
        row0 = s * ROWS_T

        pltpu.sync_copy(zeros.at[pl.ds(row0, ROWS_T)], acc.at[pl.ds(row0, ROWS_T)])
        pltpu.sync_copy(src3.at[wid], src_v)
        pltpu.sync_copy(dst3.at[wid], dst_v)
        plsc.subcore_barrier()
        _ring_loop(rows, acc, src_v, dst_v, bufs, gsem, ssem, B, U, G, NCH)
        plsc.subcore_barrier()
        pltpu.sync_copy(
            acc.at[pl.ds(row0, ROWS_T)],
            out.at[pl.ds(c * N_PAD + row0, ROWS_T)],
        )

    return spass


ROWS_W = N_PAD // NW


@functools.lru_cache(maxsize=None)
def _make_fused(d):
    B, U, G = _CHUNK_CFG[d]
    NCH = E_W // B
    R3C = 160
    assert NCH % U == 0 and ROWS_W % R3C == 0 and B >= R3C
    mesh = plsc.VectorSubcoreMesh(
        core_axis_name="c", subcore_axis_name="s", num_cores=NC, num_subcores=NS
    )

    @functools.partial(
        pl.kernel,
        out_type=(jax.ShapeDtypeStruct((NC * N_PAD, d), jnp.float32),
                  jax.ShapeDtypeStruct((N_PAD, d), jnp.float32)),
        mesh=mesh,
        compiler_params=pltpu.CompilerParams(
            use_tc_tiling_on_sc=False, needs_layout_passes=False
        ),
        scratch_types=[
            pltpu.VMEM((NCH, B), jnp.int32),
            pltpu.VMEM((NCH, B), jnp.int32),
            *([pltpu.VMEM((B, d), jnp.float32)] * U),
            *([pltpu.SemaphoreType.DMA] * U),
            *([pltpu.SemaphoreType.DMA] * U),
            pltpu.VMEM((ROWS_W,), jnp.float32),
            pltpu.SemaphoreType.REGULAR,
            pltpu.VMEM_SHARED((N_PAD, d), jnp.float32),
        ],
    )
    def fused(src3, dst3, u, dinv, zeros, out, u2, src_v, dst_v, *rest):
        bufs = rest[:U]
        gsem = rest[U:2 * U]
        ssem = rest[2 * U:3 * U]
        dinv_v = rest[3 * U]
        xsem = rest[3 * U + 1]
        acc = rest[3 * U + 2]
        c = lax.axis_index("c")
        s = lax.axis_index("s")
        wid = c * NS + s
        row0 = s * ROWS_T
        grow0 = wid * ROWS_W

        def cross_core_barrier():
            plsc.subcore_barrier()
            pl.semaphore_signal(xsem, 1, core_index=1 - c)
            pl.semaphore_wait(xsem, 1)

        pltpu.sync_copy(zeros.at[pl.ds(row0, ROWS_T)], acc.at[pl.ds(row0, ROWS_T)])
        pltpu.sync_copy(dinv.at[pl.ds(grow0, ROWS_W)], dinv_v)
        pltpu.sync_copy(src3.at[wid], src_v)
        pltpu.sync_copy(dst3.at[wid], dst_v)
        plsc.subcore_barrier()
        _ring_loop(u, acc, src_v, dst_v, bufs, gsem, ssem, B, U, G, NCH)
        plsc.subcore_barrier()
        pltpu.sync_copy(
            acc.at[pl.ds(row0, ROWS_T)],
            out.at[pl.ds(c * N_PAD + row0, ROWS_T)],
        )
        cross_core_barrier()

        for t in range(ROWS_W // R3C):
            r0 = grow0 + t * R3C
            cps = [
                pltpu.make_async_copy(out.at[pl.ds(r0, R3C)],
                                      bufs[0].at[pl.ds(0, R3C)], gsem[0]),
                pltpu.make_async_copy(out.at[pl.ds(N_PAD + r0, R3C)],
                                      bufs[1].at[pl.ds(0, R3C)], gsem[1]),
                pltpu.make_async_copy(u.at[pl.ds(r0, R3C)],
                                      bufs[2].at[pl.ds(0, R3C)], gsem[2]),
            ]
            for cp in cps:
                cp.start()
            for cp in cps:
                cp.wait()

            def rowbody(r, carry, _t=t):
                sp = plsc.load_gather(
                    dinv_v, [jnp.full((16,), _t * R3C + r, jnp.int32)]
                )
                sq = sp * sp
                for j in range(d // 16):
                    sl = pl.ds(j * 16, 16)
                    bufs[3][r, sl] = sq * (
                        bufs[0][r, sl] + bufs[1][r, sl] + bufs[2][r, sl]
                    )
                return carry

            lax.fori_loop(0, R3C, rowbody, 0)
            pltpu.sync_copy(bufs[3].at[pl.ds(0, R3C)], u2.at[pl.ds(r0, R3C)])

        cross_core_barrier()

        pltpu.sync_copy(zeros.at[pl.ds(row0, ROWS_T)], acc.at[pl.ds(row0, ROWS_T)])
        plsc.subcore_barrier()
        _ring_loop(u2, acc, src_v, dst_v, bufs, gsem, ssem, B, U, G, NCH)
        plsc.subcore_barrier()
        pltpu.sync_copy(
            acc.at[pl.ds(row0, ROWS_T)],
            out.at[pl.ds(c * N_PAD + row0, ROWS_T)],
        )

    return fused


def _row_spec(d, shift=0):
    return pl.BlockSpec((BN, d), lambda i, _s=shift: (i + _s, 0))


def _full_spec(shape):
    nd = len(shape)
    return pl.BlockSpec(shape, lambda i, _nd=nd: (0,) * nd)


def _tc_call(body, in_specs, out_shapes, out_specs):
    return pl.pallas_call(
        body,
        grid=(NB,),
        in_specs=in_specs,
        out_shape=out_shapes,
        out_specs=out_specs,
    )


def _tc_a(degp, x, w1):
    def body(p0_ref, p1_ref, x_ref, w1_ref, dinv_ref, u_ref):
        deg = 1.0 + p0_ref[:, 0:1] + p1_ref[:, 0:1]
        dinv = lax.rsqrt(deg)
        y1 = jnp.dot(x_ref[...], w1_ref[...], preferred_element_type=jnp.float32)
        dinv_ref[...] = dinv
        u_ref[...] = dinv * y1

    return _tc_call(
        body,
        [_row_spec(16), _row_spec(16, NB), _row_spec(D_IN), _full_spec((D_IN, D_HID))],
        (jax.ShapeDtypeStruct((N_PAD, 1), jnp.float32),
         jax.ShapeDtypeStruct((N_PAD, D_HID), jnp.float32)),
        (_row_spec(1), _row_spec(D_HID)),
    )(degp, degp, x, w1)


def _tc_call(body, in_specs, out_shapes, out_specs):
    return pl.pallas_call(
        body,
        grid=(NB,),
        in_specs=in_specs,
        out_shape=out_shapes,
        out_specs=out_specs,
    )


def _tc_a(degp, x, w1):
    def body(p0_ref, p1_ref, x_ref, w1_ref, dinv_ref, u_ref):
        deg = 1.0 + p0_ref[:, 0:1] + p1_ref[:, 0:1]
        dinv = lax.rsqrt(deg)
        y1 = jnp.dot(x_ref[...], w1_ref[...], preferred_element_type=jnp.float32)
        dinv_ref[...] = dinv
        u_ref[...] = dinv * y1

    return _tc_call(
        body,
        [_row_spec(16), _row_spec(16, NB), _row_spec(D_IN), _full_spec((D_IN, D_HID))],
        (jax.ShapeDtypeStruct((N_PAD, 1), jnp.float32),
         jax.ShapeDtypeStruct((N_PAD, D_HID), jnp.float32)),
        (_row_spec(1), _row_spec(D_HID)),
    )(degp, degp, x, w1)


def _tc_c(s, u2, dinv, b1, w2p):
    def body(s0_ref, s1_ref, u2_ref, dinv_ref, b1_ref, w2_ref, o_ref):
        dinv = dinv_ref[...]
        t = dinv * (s0_ref[...] + s1_ref[...] + u2_ref[...])
        h = jnp.maximum(t + b1_ref[...], 0.0)
        y2 = jnp.dot(h, w2_ref[...], preferred_element_type=jnp.float32)
        o_ref[...] = dinv * y2

    return _tc_call(
        body,
        [_row_spec(D_HID), _row_spec(D_HID, NB), _row_spec(D_HID), _row_spec(1),
         _full_spec((1, D_HID)), _full_spec((D_HID, 16))],
        jax.ShapeDtypeStruct((N_PAD, 16), jnp.float32),
        _row_spec(16),
    )(s, s, u2, dinv, b1, w2p)


def _tc_e(s, u4, dinv, b2p):
    def body(s0_ref, s1_ref, u4_ref, dinv_ref, b2_ref, o_ref):
        dinv = dinv_ref[...]
        o = dinv * (s0_ref[...] + s1_ref[...] + u4_ref[...]) + b2_ref[...]
        cols = lax.broadcasted_iota(jnp.int32, (BN, 16), 1)
        valid = cols < D_OUT
        m = jnp.max(jnp.where(valid, o, -jnp.inf), axis=1, keepdims=True)
        e = jnp.where(valid, jnp.exp(o - m), 0.0)
        lse = m + jnp.log(jnp.sum(e, axis=1, keepdims=True))
        o_ref[...] = o - lse

    return _tc_call(
        body,
        [_row_spec(16), _row_spec(16, NB), _row_spec(16), _row_spec(1),
         _full_spec((1, 16))],
        jax.ShapeDtypeStruct((N_PAD, 16), jnp.float32),
        _row_spec(16),
    )(s, s, u4, dinv, b2p)


def kernel(x, edge_index, W1, b1, W2, b2):
    def idx3(d):
        B, _, _ = _CHUNK_CFG[d]
        shape = (NW, E_W // B, B)
        return edge_index[0].reshape(shape), edge_index[1].reshape(shape)

    src64, dst64 = idx3(64)
    src16, dst16 = idx3(16)
    xp = jnp.zeros((N_PAD, D_IN), jnp.float32).at[:N].set(x)
    zeros64 = jnp.zeros((N_PAD, D_HID), jnp.float32)
    zeros16 = jnp.zeros((N_PAD, 16), jnp.float32)
    ones16 = jnp.ones((N_PAD, 16), jnp.float32)
    w2p = jnp.zeros((D_HID, 16), jnp.float32).at[:, :D_OUT].set(W2)
    b2p = jnp.zeros((1, 16), jnp.float32).at[0, :D_OUT].set(b2)

    spass16 = _make_spass(16)
    fused64 = _make_fused(D_HID)
    fused16 = _make_fused(16)

    degp = spass16(src16, dst16, ones16, zeros16)
    dinv, u = _tc_a(degp, xp, W1)
    dinv1d = dinv.reshape(N_PAD)
    s2, u2 = fused64(src64, dst64, u, dinv1d, zeros64)
    u3 = _tc_c(s2, u2, dinv, b1.reshape(1, D_HID), w2p)
    s4, u4 = fused16(src16, dst16, u3, dinv1d, zeros16)
    out = _tc_e(s4, u4, dinv, b2p)
    return out[:N, :D_OUT]
